# baseline (speedup 1.0000x reference)
import functools

import jax
import jax.numpy as jnp
from jax import lax
from jax.experimental import pallas as pl
from jax.experimental.pallas import tpu as pltpu
from jax.experimental.pallas import tpu_sc as plsc

N = 10000
E = 320000
D = 128
BN_SCALE = float(1.0 / (1.0 + 1e-5) ** 0.5)

NC, NS = 2, 16
TILES = NC * NS
CH = 128
NCH = 80
SB = 8
EP = TILES * NCH * CH
NP = 10240
RPT = NP // NS

_mesh = plsc.VectorSubcoreMesh(core_axis_name="c", subcore_axis_name="s")


def _deg_body(dst_hbm, zeros_hbm, ones_hbm, out_hbm, idx_d, ones_v, acc, sem):
    c = lax.axis_index("c")
    s = lax.axis_index("s")
    w = c * NS + s
    pltpu.sync_copy(zeros_hbm.at[pl.ds(s * RPT, RPT)], acc.at[pl.ds(s * RPT, RPT)])
    pltpu.sync_copy(ones_hbm, ones_v)
    plsc.subcore_barrier()

    def sblock(b, carry):
        pltpu.sync_copy(dst_hbm.at[w].at[b], idx_d)

        def body(j, carry2):
            pltpu.sync_copy(ones_v, acc.at[idx_d.at[j]], add=True)
            return carry2

        lax.fori_loop(0, SB, body, 0)
        return carry

    lax.fori_loop(0, NCH // SB, sblock, 0)
    plsc.subcore_barrier()
    pltpu.sync_copy(acc.at[pl.ds(s * RPT, RPT)], out_hbm.at[c, pl.ds(s * RPT, RPT)])


_sc_deg = pl.kernel(
    _deg_body,
    out_type=jax.ShapeDtypeStruct((NC, NP, D), jnp.float32),
    mesh=_mesh,
    scratch_types=[
        pltpu.VMEM((SB, CH), jnp.int32),
        pltpu.VMEM((CH, D), jnp.float32),
        pltpu.VMEM_SHARED((NP, D), jnp.float32),
        pltpu.SemaphoreType.DMA,
    ],
)


def _edge_body(u_hbm, src_hbm, dst_hbm, zeros_hbm, out_hbm,
               idx_s, idx_d, rows0, rows1, acc, sem0, sem1):
    c = lax.axis_index("c")
    s = lax.axis_index("s")
    w = c * NS + s
    pltpu.sync_copy(zeros_hbm.at[pl.ds(s * RPT, RPT)], acc.at[pl.ds(s * RPT, RPT)])
    plsc.subcore_barrier()

    rows = (rows0, rows1)
    sems = (sem0, sem1)

    def sblock(b, carry):
        pltpu.sync_copy(src_hbm.at[w].at[b], idx_s)
        pltpu.sync_copy(dst_hbm.at[w].at[b], idx_d)
        cps = [None] * SB
        cps[0] = pltpu.async_copy(u_hbm.at[idx_s.at[0]], rows[0], sems[0])
        for j in range(SB):
            if j + 1 < SB:
                cps[j + 1] = pltpu.async_copy(
                    u_hbm.at[idx_s.at[j + 1]], rows[(j + 1) % 2], sems[(j + 1) % 2])
            cps[j].wait()
            pltpu.sync_copy(rows[j % 2], acc.at[idx_d.at[j]], add=True)
        return carry

    lax.fori_loop(0, NCH // SB, sblock, 0)
    plsc.subcore_barrier()
    pltpu.sync_copy(acc.at[pl.ds(s * RPT, RPT)], out_hbm.at[c, pl.ds(s * RPT, RPT)])


_sc_edges = pl.kernel(
    _edge_body,
    out_type=jax.ShapeDtypeStruct((NC, NP, D), jnp.float32),
    mesh=_mesh,
    scratch_types=[
        pltpu.VMEM((SB, CH), jnp.int32),
        pltpu.VMEM((SB, CH), jnp.int32),
        pltpu.VMEM((CH, D), jnp.float32),
        pltpu.VMEM((CH, D), jnp.float32),
        pltpu.VMEM_SHARED((NP, D), jnp.float32),
        pltpu.SemaphoreType.DMA,
        pltpu.SemaphoreType.DMA,
    ],
)


BR = 1024


def _dinv(h_ref):
    deg = 1.0 + h_ref[0, :, 0] + h_ref[1, :, 0]
    return lax.rsqrt(deg)[:, None]


def _pre_body(x_ref, w_ref, h_ref, o_ref):
    xw = jnp.dot(x_ref[...], w_ref[...], preferred_element_type=jnp.float32)
    o_ref[...] = xw * _dinv(h_ref)


def _mid_body(s_ref, u_ref, h_ref, b_ref, g_ref, be_ref, w_ref, o_ref):
    dinv = _dinv(h_ref)
    pre = dinv * (s_ref[0] + s_ref[1] + u_ref[...]) + b_ref[...]
    h = jnp.maximum(pre * (g_ref[...] * BN_SCALE) + be_ref[...], 0.0)
    o_ref[...] = jnp.dot(h, w_ref[...], preferred_element_type=jnp.float32) * dinv


def _fin_body(s_ref, u_ref, h_ref, b_ref, o_ref):
    o_ref[...] = _dinv(h_ref) * (s_ref[0] + s_ref[1] + u_ref[...]) + b_ref[...]


_GRID = NP // BR
_bs_rows = pl.BlockSpec((BR, D), lambda i: (i, 0))
_bs_part = pl.BlockSpec((NC, BR, D), lambda i: (0, i, 0))
_bs_hist = pl.BlockSpec((NC, BR, D), lambda i: (0, i, 0))
_bs_w = pl.BlockSpec((D, D), lambda i: (0, 0))
_bs_vec = pl.BlockSpec((1, D), lambda i: (0, 0))

_tc_pre = pl.pallas_call(
    _pre_body,
    grid=(_GRID,),
    in_specs=[_bs_rows, _bs_w, _bs_hist],
    out_specs=_bs_rows,
    out_shape=jax.ShapeDtypeStruct((NP, D), jnp.float32),
)

_tc_mid = pl.pallas_call(
    _mid_body,
    grid=(_GRID,),
    in_specs=[_bs_part, _bs_rows, _bs_hist, _bs_vec, _bs_vec, _bs_vec, _bs_w],
    out_specs=_bs_rows,
    out_shape=jax.ShapeDtypeStruct((NP, D), jnp.float32),
)

_tc_fin = pl.pallas_call(
    _fin_body,
    grid=(_GRID,),
    in_specs=[_bs_part, _bs_rows, _bs_hist, _bs_vec],
    out_specs=_bs_rows,
    out_shape=jax.ShapeDtypeStruct((NP, D), jnp.float32),
)


@jax.jit
def kernel(x, edge_index, W1, b1, g1, be1, W2, b2, g2, be2, W3, b3):
    f32 = jnp.float32
    xp = jnp.zeros((NP, D), f32).at[:N].set(x)
    pad = jnp.full((EP - E,), N, jnp.int32)
    srcp = jnp.concatenate([edge_index[0], pad]).reshape(TILES, NCH // SB, SB, CH)
    dstp = jnp.concatenate([edge_index[1], pad]).reshape(TILES, NCH // SB, SB, CH)
    zeros = jnp.zeros((NP, D), f32)
    onesr = jnp.ones((CH, D), f32)
    b1r, g1r, be1r = b1.reshape(1, D), g1.reshape(1, D), be1.reshape(1, D)
    b2r, g2r, be2r = b2.reshape(1, D), g2.reshape(1, D), be2.reshape(1, D)
    b3r = b3.reshape(1, D)

    hist = _sc_deg(dstp, zeros, onesr)
    u1 = _tc_pre(xp, W1, hist)
    s1 = _sc_edges(u1, srcp, dstp, zeros)
    u2 = _tc_mid(s1, u1, hist, b1r, g1r, be1r, W2)
    s2 = _sc_edges(u2, srcp, dstp, zeros)
    u3 = _tc_mid(s2, u2, hist, b2r, g2r, be2r, W3)
    s3 = _sc_edges(u3, srcp, dstp, zeros)
    outp = _tc_fin(s3, u3, hist, b3r)
    return outp[:N]


# speedup vs baseline: 2.9506x; 2.9506x over previous
import functools

import jax
import jax.numpy as jnp
from jax import lax
from jax.experimental import pallas as pl
from jax.experimental.pallas import tpu as pltpu
from jax.experimental.pallas import tpu_sc as plsc

N = 10000
E = 320000
D = 128
BN_SCALE = float(1.0 / (1.0 + 1e-5) ** 0.5)

NC, NS = 2, 16
TILES = NC * NS
CH = 128
NCH = 80
SB = 8
EP = TILES * NCH * CH
NP = 10240
RPT = NP // NS

_mesh = plsc.VectorSubcoreMesh(core_axis_name="c", subcore_axis_name="s")


def _deg_body(dst_hbm, zeros_hbm, ones_hbm, out_hbm, idx_d, ones_v, acc, sem):
    c = lax.axis_index("c")
    s = lax.axis_index("s")
    w = c * NS + s
    pltpu.sync_copy(zeros_hbm.at[pl.ds(s * RPT, RPT)], acc.at[pl.ds(s * RPT, RPT)])
    pltpu.sync_copy(ones_hbm, ones_v)
    plsc.subcore_barrier()

    def sblock(b, carry):
        pltpu.sync_copy(dst_hbm.at[w].at[b], idx_d)

        def body(j, carry2):
            pltpu.sync_copy(ones_v, acc.at[idx_d.at[j]], add=True)
            return carry2

        lax.fori_loop(0, SB, body, 0)
        return carry

    lax.fori_loop(0, NCH // SB, sblock, 0)
    plsc.subcore_barrier()
    pltpu.sync_copy(acc.at[pl.ds(s * RPT, RPT)], out_hbm.at[c, pl.ds(s * RPT, RPT)])


_sc_deg = pl.kernel(
    _deg_body,
    out_type=jax.ShapeDtypeStruct((NC, NP, D), jnp.float32),
    mesh=_mesh,
    scratch_types=[
        pltpu.VMEM((SB, CH), jnp.int32),
        pltpu.VMEM((CH, D), jnp.float32),
        pltpu.VMEM_SHARED((NP, D), jnp.float32),
        pltpu.SemaphoreType.DMA,
    ],
)


def _edge_body(u_hbm, src_hbm, dst_hbm, zeros_hbm, out_hbm,
               idx_s, idx_d, rows0, rows1, acc, sem0, sem1):
    c = lax.axis_index("c")
    s = lax.axis_index("s")
    w = c * NS + s
    pltpu.sync_copy(zeros_hbm.at[pl.ds(s * RPT, RPT)], acc.at[pl.ds(s * RPT, RPT)])
    plsc.subcore_barrier()

    rows = (rows0, rows1)
    sems = (sem0, sem1)

    def sblock(b, carry):
        pltpu.sync_copy(src_hbm.at[w].at[b], idx_s)
        pltpu.sync_copy(dst_hbm.at[w].at[b], idx_d)
        cps = [None] * SB
        cps[0] = pltpu.async_copy(u_hbm.at[idx_s.at[0]], rows[0], sems[0])
        for j in range(SB):
            if j + 1 < SB:
                cps[j + 1] = pltpu.async_copy(
                    u_hbm.at[idx_s.at[j + 1]], rows[(j + 1) % 2], sems[(j + 1) % 2])
            cps[j].wait()
            pltpu.sync_copy(rows[j % 2], acc.at[idx_d.at[j]], add=True)
        return carry

    lax.fori_loop(0, NCH // SB, sblock, 0)
    plsc.subcore_barrier()
    pltpu.sync_copy(acc.at[pl.ds(s * RPT, RPT)], out_hbm.at[c, pl.ds(s * RPT, RPT)])


_sc_edges = pl.kernel(
    _edge_body,
    out_type=jax.ShapeDtypeStruct((NC, NP, D), jnp.float32),
    mesh=_mesh,
    scratch_types=[
        pltpu.VMEM((SB, CH), jnp.int32),
        pltpu.VMEM((SB, CH), jnp.int32),
        pltpu.VMEM((CH, D), jnp.float32),
        pltpu.VMEM((CH, D), jnp.float32),
        pltpu.VMEM_SHARED((NP, D), jnp.float32),
        pltpu.SemaphoreType.DMA,
        pltpu.SemaphoreType.DMA,
    ],
)


BR = 1024


def _dinv(h_ref):
    deg = 1.0 + h_ref[0, :, 0] + h_ref[1, :, 0]
    return lax.rsqrt(deg)[:, None]


def _pre_body(x_ref, w_ref, h_ref, o_ref):
    xw = jnp.dot(x_ref[...], w_ref[...], preferred_element_type=jnp.float32)
    o_ref[...] = xw * _dinv(h_ref)


def _mid_body(s_ref, u_ref, h_ref, b_ref, g_ref, be_ref, w_ref, o_ref):
    dinv = _dinv(h_ref)
    pre = dinv * (s_ref[0] + s_ref[1] + u_ref[...]) + b_ref[...]
    h = jnp.maximum(pre * (g_ref[...] * BN_SCALE) + be_ref[...], 0.0)
    o_ref[...] = jnp.dot(h, w_ref[...], preferred_element_type=jnp.float32) * dinv


def _fin_body(s_ref, u_ref, h_ref, b_ref, o_ref):
    o_ref[...] = _dinv(h_ref) * (s_ref[0] + s_ref[1] + u_ref[...]) + b_ref[...]


_GRID = NP // BR
_bs_rows = pl.BlockSpec((BR, D), lambda i: (i, 0))
_bs_part = pl.BlockSpec((NC, BR, D), lambda i: (0, i, 0))
_bs_hist = pl.BlockSpec((NC, BR, D), lambda i: (0, i, 0))
_bs_w = pl.BlockSpec((D, D), lambda i: (0, 0))
_bs_vec = pl.BlockSpec((1, D), lambda i: (0, 0))

_tc_pre = pl.pallas_call(
    _pre_body,
    grid=(_GRID,),
    in_specs=[_bs_rows, _bs_w, _bs_hist],
    out_specs=_bs_rows,
    out_shape=jax.ShapeDtypeStruct((NP, D), jnp.float32),
)

_tc_mid = pl.pallas_call(
    _mid_body,
    grid=(_GRID,),
    in_specs=[_bs_part, _bs_rows, _bs_hist, _bs_vec, _bs_vec, _bs_vec, _bs_w],
    out_specs=_bs_rows,
    out_shape=jax.ShapeDtypeStruct((NP, D), jnp.float32),
)

_tc_fin = pl.pallas_call(
    _fin_body,
    grid=(_GRID,),
    in_specs=[_bs_part, _bs_rows, _bs_hist, _bs_vec],
    out_specs=_bs_rows,
    out_shape=jax.ShapeDtypeStruct((NP, D), jnp.float32),
)


@jax.jit
def kernel(x, edge_index, W1, b1, g1, be1, W2, b2, g2, be2, W3, b3):
    f32 = jnp.float32
    xp = jnp.zeros((NP, D), f32).at[:N].set(x)
    pad = N + jnp.arange(EP - E, dtype=jnp.int32) % (NP - N)
    srcp = jnp.concatenate([edge_index[0], pad]).reshape(TILES, NCH // SB, SB, CH)
    dstp = jnp.concatenate([edge_index[1], pad]).reshape(TILES, NCH // SB, SB, CH)
    zeros = jnp.zeros((NP, D), f32)
    onesr = jnp.ones((CH, D), f32)
    b1r, g1r, be1r = b1.reshape(1, D), g1.reshape(1, D), be1.reshape(1, D)
    b2r, g2r, be2r = b2.reshape(1, D), g2.reshape(1, D), be2.reshape(1, D)
    b3r = b3.reshape(1, D)

    hist = _sc_deg(dstp, zeros, onesr)
    u1 = _tc_pre(xp, W1, hist)
    s1 = _sc_edges(u1, srcp, dstp, zeros)
    u2 = _tc_mid(s1, u1, hist, b1r, g1r, be1r, W2)
    s2 = _sc_edges(u2, srcp, dstp, zeros)
    u3 = _tc_mid(s2, u2, hist, b2r, g2r, be2r, W3)
    s3 = _sc_edges(u3, srcp, dstp, zeros)
    outp = _tc_fin(s3, u3, hist, b3r)
    return outp[:N]


# speedup vs baseline: 2.9527x; 1.0007x over previous
import functools

import jax
import jax.numpy as jnp
from jax import lax
from jax.experimental import pallas as pl
from jax.experimental.pallas import tpu as pltpu
from jax.experimental.pallas import tpu_sc as plsc

N = 10000
E = 320000
D = 128
BN_SCALE = float(1.0 / (1.0 + 1e-5) ** 0.5)

NC, NS = 2, 16
TILES = NC * NS
CH = 128
NCH = 80
SB = 8
EP = TILES * NCH * CH
NP = 10240
RPT = NP // NS

_mesh = plsc.VectorSubcoreMesh(core_axis_name="c", subcore_axis_name="s")


def _deg_body(dst_hbm, zeros_hbm, ones_hbm, out_hbm, idx_d, ones_v, acc,
              sem0, sem1):
    c = lax.axis_index("c")
    s = lax.axis_index("s")
    w = c * NS + s
    pltpu.sync_copy(zeros_hbm.at[pl.ds(s * RPT, RPT)], acc.at[pl.ds(s * RPT, RPT)])
    pltpu.sync_copy(ones_hbm, ones_v)
    plsc.subcore_barrier()
    sems = (sem0, sem1)

    def sblock(b, carry):
        pltpu.sync_copy(dst_hbm.at[w].at[b], idx_d)
        cs = [None] * SB
        for j in range(SB):
            if j >= 2:
                cs[j - 2].wait()
            cs[j] = pltpu.async_copy(ones_v, acc.at[idx_d.at[j]],
                                     sems[j % 2], add=True)
        cs[SB - 2].wait()
        cs[SB - 1].wait()
        return carry

    lax.fori_loop(0, NCH // SB, sblock, 0)
    plsc.subcore_barrier()
    pltpu.sync_copy(acc.at[pl.ds(s * RPT, RPT)], out_hbm.at[c, pl.ds(s * RPT, RPT)])


_sc_deg = pl.kernel(
    _deg_body,
    out_type=jax.ShapeDtypeStruct((NC, NP, D), jnp.float32),
    mesh=_mesh,
    scratch_types=[
        pltpu.VMEM((SB, CH), jnp.int32),
        pltpu.VMEM((CH, D), jnp.float32),
        pltpu.VMEM_SHARED((NP, D), jnp.float32),
        pltpu.SemaphoreType.DMA,
        pltpu.SemaphoreType.DMA,
    ],
)


def _edge_body(u_hbm, src_hbm, dst_hbm, zeros_hbm, out_hbm,
               idx_s, idx_d, rows0, rows1, acc, gsem0, gsem1, ssem0, ssem1):
    c = lax.axis_index("c")
    s = lax.axis_index("s")
    w = c * NS + s
    pltpu.sync_copy(zeros_hbm.at[pl.ds(s * RPT, RPT)], acc.at[pl.ds(s * RPT, RPT)])
    plsc.subcore_barrier()

    rows = (rows0, rows1)
    gsems = (gsem0, gsem1)
    ssems = (ssem0, ssem1)

    def sblock(b, carry):
        pltpu.sync_copy(src_hbm.at[w].at[b], idx_s)
        pltpu.sync_copy(dst_hbm.at[w].at[b], idx_d)
        cg = [None] * SB
        cs = [None] * SB
        cg[0] = pltpu.async_copy(u_hbm.at[idx_s.at[0]], rows[0], gsems[0])
        for j in range(SB):
            if j >= 1:
                cs[j - 1].wait()
            if j + 1 < SB:
                cg[j + 1] = pltpu.async_copy(
                    u_hbm.at[idx_s.at[j + 1]], rows[(j + 1) % 2], gsems[(j + 1) % 2])
            cg[j].wait()
            cs[j] = pltpu.async_copy(
                rows[j % 2], acc.at[idx_d.at[j]], ssems[j % 2], add=True)
        cs[SB - 1].wait()
        return carry

    lax.fori_loop(0, NCH // SB, sblock, 0)
    plsc.subcore_barrier()
    pltpu.sync_copy(acc.at[pl.ds(s * RPT, RPT)], out_hbm.at[c, pl.ds(s * RPT, RPT)])


_sc_edges = pl.kernel(
    _edge_body,
    out_type=jax.ShapeDtypeStruct((NC, NP, D), jnp.float32),
    mesh=_mesh,
    scratch_types=[
        pltpu.VMEM((SB, CH), jnp.int32),
        pltpu.VMEM((SB, CH), jnp.int32),
        pltpu.VMEM((CH, D), jnp.float32),
        pltpu.VMEM((CH, D), jnp.float32),
        pltpu.VMEM_SHARED((NP, D), jnp.float32),
        pltpu.SemaphoreType.DMA,
        pltpu.SemaphoreType.DMA,
        pltpu.SemaphoreType.DMA,
        pltpu.SemaphoreType.DMA,
    ],
)


BR = 1024


def _dinv(h_ref):
    deg = 1.0 + h_ref[0, :, 0] + h_ref[1, :, 0]
    return lax.rsqrt(deg)[:, None]


def _pre_body(x_ref, w_ref, h_ref, o_ref):
    xw = jnp.dot(x_ref[...], w_ref[...], preferred_element_type=jnp.float32)
    o_ref[...] = xw * _dinv(h_ref)


def _mid_body(s_ref, u_ref, h_ref, b_ref, g_ref, be_ref, w_ref, o_ref):
    dinv = _dinv(h_ref)
    pre = dinv * (s_ref[0] + s_ref[1] + u_ref[...]) + b_ref[...]
    h = jnp.maximum(pre * (g_ref[...] * BN_SCALE) + be_ref[...], 0.0)
    o_ref[...] = jnp.dot(h, w_ref[...], preferred_element_type=jnp.float32) * dinv


def _fin_body(s_ref, u_ref, h_ref, b_ref, o_ref):
    o_ref[...] = _dinv(h_ref) * (s_ref[0] + s_ref[1] + u_ref[...]) + b_ref[...]


_GRID = NP // BR
_bs_rows = pl.BlockSpec((BR, D), lambda i: (i, 0))
_bs_part = pl.BlockSpec((NC, BR, D), lambda i: (0, i, 0))
_bs_hist = pl.BlockSpec((NC, BR, D), lambda i: (0, i, 0))
_bs_w = pl.BlockSpec((D, D), lambda i: (0, 0))
_bs_vec = pl.BlockSpec((1, D), lambda i: (0, 0))

_tc_pre = pl.pallas_call(
    _pre_body,
    grid=(_GRID,),
    in_specs=[_bs_rows, _bs_w, _bs_hist],
    out_specs=_bs_rows,
    out_shape=jax.ShapeDtypeStruct((NP, D), jnp.float32),
)

_tc_mid = pl.pallas_call(
    _mid_body,
    grid=(_GRID,),
    in_specs=[_bs_part, _bs_rows, _bs_hist, _bs_vec, _bs_vec, _bs_vec, _bs_w],
    out_specs=_bs_rows,
    out_shape=jax.ShapeDtypeStruct((NP, D), jnp.float32),
)

_tc_fin = pl.pallas_call(
    _fin_body,
    grid=(_GRID,),
    in_specs=[_bs_part, _bs_rows, _bs_hist, _bs_vec],
    out_specs=_bs_rows,
    out_shape=jax.ShapeDtypeStruct((NP, D), jnp.float32),
)


@jax.jit
def kernel(x, edge_index, W1, b1, g1, be1, W2, b2, g2, be2, W3, b3):
    f32 = jnp.float32
    xp = jnp.zeros((NP, D), f32).at[:N].set(x)
    pad = N + jnp.arange(EP - E, dtype=jnp.int32) % (NP - N)
    srcp = jnp.concatenate([edge_index[0], pad]).reshape(TILES, NCH // SB, SB, CH)
    dstp = jnp.concatenate([edge_index[1], pad]).reshape(TILES, NCH // SB, SB, CH)
    zeros = jnp.zeros((NP, D), f32)
    onesr = jnp.ones((CH, D), f32)
    b1r, g1r, be1r = b1.reshape(1, D), g1.reshape(1, D), be1.reshape(1, D)
    b2r, g2r, be2r = b2.reshape(1, D), g2.reshape(1, D), be2.reshape(1, D)
    b3r = b3.reshape(1, D)

    hist = _sc_deg(dstp, zeros, onesr)
    u1 = _tc_pre(xp, W1, hist)
    s1 = _sc_edges(u1, srcp, dstp, zeros)
    u2 = _tc_mid(s1, u1, hist, b1r, g1r, be1r, W2)
    s2 = _sc_edges(u2, srcp, dstp, zeros)
    u3 = _tc_mid(s2, u2, hist, b2r, g2r, be2r, W3)
    s3 = _sc_edges(u3, srcp, dstp, zeros)
    outp = _tc_fin(s3, u3, hist, b3r)
    return outp[:N]


# speedup vs baseline: 3.1586x; 1.0697x over previous
import functools

import jax
import jax.numpy as jnp
from jax import lax
from jax.experimental import pallas as pl
from jax.experimental.pallas import tpu as pltpu
from jax.experimental.pallas import tpu_sc as plsc

N = 10000
E = 320000
D = 128
BN_SCALE = float(1.0 / (1.0 + 1e-5) ** 0.5)

NC, NS = 2, 16
TILES = NC * NS
CH = 128
NCH = 80
SB = 16
EP = TILES * NCH * CH
NP = 10240
RPT = NP // NS

_mesh = plsc.VectorSubcoreMesh(core_axis_name="c", subcore_axis_name="s")


def _deg_body(dst_hbm, zeros_hbm, ones_hbm, out_hbm, idx_d, ones_v, acc,
              sem0, sem1):
    c = lax.axis_index("c")
    s = lax.axis_index("s")
    w = c * NS + s
    pltpu.sync_copy(zeros_hbm.at[pl.ds(s * RPT, RPT)], acc.at[pl.ds(s * RPT, RPT)])
    pltpu.sync_copy(ones_hbm, ones_v)
    plsc.subcore_barrier()
    sems = (sem0, sem1)

    def sblock(b, carry):
        pltpu.sync_copy(dst_hbm.at[w].at[b], idx_d)
        cs = [None] * SB
        for j in range(SB):
            if j >= 2:
                cs[j - 2].wait()
            cs[j] = pltpu.async_copy(ones_v, acc.at[idx_d.at[j]],
                                     sems[j % 2], add=True)
        cs[SB - 2].wait()
        cs[SB - 1].wait()
        return carry

    lax.fori_loop(0, NCH // SB, sblock, 0)
    plsc.subcore_barrier()
    pltpu.sync_copy(acc.at[pl.ds(s * RPT, RPT)], out_hbm.at[c, pl.ds(s * RPT, RPT)])


_sc_deg = pl.kernel(
    _deg_body,
    out_type=jax.ShapeDtypeStruct((NC, NP, D), jnp.float32),
    mesh=_mesh,
    scratch_types=[
        pltpu.VMEM((SB, CH), jnp.int32),
        pltpu.VMEM((CH, D), jnp.float32),
        pltpu.VMEM_SHARED((NP, D), jnp.float32),
        pltpu.SemaphoreType.DMA,
        pltpu.SemaphoreType.DMA,
    ],
)


def _edge_body(u_hbm, src_hbm, dst_hbm, zeros_hbm, out_hbm,
               idx_s, idx_d, rows0, rows1, acc, gsem0, gsem1, ssem0, ssem1):
    c = lax.axis_index("c")
    s = lax.axis_index("s")
    w = c * NS + s
    pltpu.sync_copy(zeros_hbm.at[pl.ds(s * RPT, RPT)], acc.at[pl.ds(s * RPT, RPT)])
    plsc.subcore_barrier()

    rows = (rows0, rows1)
    gsems = (gsem0, gsem1)
    ssems = (ssem0, ssem1)

    def sblock(b, carry):
        pltpu.sync_copy(src_hbm.at[w].at[b], idx_s)
        pltpu.sync_copy(dst_hbm.at[w].at[b], idx_d)
        cg = [None] * SB
        cs = [None] * SB
        cg[0] = pltpu.async_copy(u_hbm.at[idx_s.at[0]], rows[0], gsems[0])
        for j in range(SB):
            if j >= 1:
                cs[j - 1].wait()
            if j + 1 < SB:
                cg[j + 1] = pltpu.async_copy(
                    u_hbm.at[idx_s.at[j + 1]], rows[(j + 1) % 2], gsems[(j + 1) % 2])
            cg[j].wait()
            cs[j] = pltpu.async_copy(
                rows[j % 2], acc.at[idx_d.at[j]], ssems[j % 2], add=True)
        cs[SB - 1].wait()
        return carry

    lax.fori_loop(0, NCH // SB, sblock, 0)
    plsc.subcore_barrier()
    pltpu.sync_copy(acc.at[pl.ds(s * RPT, RPT)], out_hbm.at[c, pl.ds(s * RPT, RPT)])


_sc_edges = pl.kernel(
    _edge_body,
    out_type=jax.ShapeDtypeStruct((NC, NP, D), jnp.float32),
    mesh=_mesh,
    scratch_types=[
        pltpu.VMEM((SB, CH), jnp.int32),
        pltpu.VMEM((SB, CH), jnp.int32),
        pltpu.VMEM((CH, D), jnp.float32),
        pltpu.VMEM((CH, D), jnp.float32),
        pltpu.VMEM_SHARED((NP, D), jnp.float32),
        pltpu.SemaphoreType.DMA,
        pltpu.SemaphoreType.DMA,
        pltpu.SemaphoreType.DMA,
        pltpu.SemaphoreType.DMA,
    ],
)


BR = 1024


def _dinv(h_ref):
    deg = 1.0 + h_ref[0, :, 0] + h_ref[1, :, 0]
    return lax.rsqrt(deg)[:, None]


def _pre_body(x_ref, w_ref, h_ref, o_ref):
    xw = jnp.dot(x_ref[...], w_ref[...], preferred_element_type=jnp.float32)
    o_ref[...] = xw * _dinv(h_ref)


def _mid_body(s_ref, u_ref, h_ref, b_ref, g_ref, be_ref, w_ref, o_ref):
    dinv = _dinv(h_ref)
    pre = dinv * (s_ref[0] + s_ref[1] + u_ref[...]) + b_ref[...]
    h = jnp.maximum(pre * (g_ref[...] * BN_SCALE) + be_ref[...], 0.0)
    o_ref[...] = jnp.dot(h, w_ref[...], preferred_element_type=jnp.float32) * dinv


def _fin_body(s_ref, u_ref, h_ref, b_ref, o_ref):
    o_ref[...] = _dinv(h_ref) * (s_ref[0] + s_ref[1] + u_ref[...]) + b_ref[...]


_GRID = NP // BR
_bs_rows = pl.BlockSpec((BR, D), lambda i: (i, 0))
_bs_part = pl.BlockSpec((NC, BR, D), lambda i: (0, i, 0))
_bs_hist = pl.BlockSpec((NC, BR, D), lambda i: (0, i, 0))
_bs_w = pl.BlockSpec((D, D), lambda i: (0, 0))
_bs_vec = pl.BlockSpec((1, D), lambda i: (0, 0))

_tc_pre = pl.pallas_call(
    _pre_body,
    grid=(_GRID,),
    in_specs=[_bs_rows, _bs_w, _bs_hist],
    out_specs=_bs_rows,
    out_shape=jax.ShapeDtypeStruct((NP, D), jnp.float32),
)

_tc_mid = pl.pallas_call(
    _mid_body,
    grid=(_GRID,),
    in_specs=[_bs_part, _bs_rows, _bs_hist, _bs_vec, _bs_vec, _bs_vec, _bs_w],
    out_specs=_bs_rows,
    out_shape=jax.ShapeDtypeStruct((NP, D), jnp.float32),
)

_tc_fin = pl.pallas_call(
    _fin_body,
    grid=(_GRID,),
    in_specs=[_bs_part, _bs_rows, _bs_hist, _bs_vec],
    out_specs=_bs_rows,
    out_shape=jax.ShapeDtypeStruct((NP, D), jnp.float32),
)


@jax.jit
def kernel(x, edge_index, W1, b1, g1, be1, W2, b2, g2, be2, W3, b3):
    f32 = jnp.float32
    xp = jnp.zeros((NP, D), f32).at[:N].set(x)
    pad = N + jnp.arange(EP - E, dtype=jnp.int32) % (NP - N)
    srcp = jnp.concatenate([edge_index[0], pad]).reshape(TILES, NCH // SB, SB, CH)
    dstp = jnp.concatenate([edge_index[1], pad]).reshape(TILES, NCH // SB, SB, CH)
    zeros = jnp.zeros((NP, D), f32)
    onesr = jnp.ones((CH, D), f32)
    b1r, g1r, be1r = b1.reshape(1, D), g1.reshape(1, D), be1.reshape(1, D)
    b2r, g2r, be2r = b2.reshape(1, D), g2.reshape(1, D), be2.reshape(1, D)
    b3r = b3.reshape(1, D)

    hist = _sc_deg(dstp, zeros, onesr)
    u1 = _tc_pre(xp, W1, hist)
    s1 = _sc_edges(u1, srcp, dstp, zeros)
    u2 = _tc_mid(s1, u1, hist, b1r, g1r, be1r, W2)
    s2 = _sc_edges(u2, srcp, dstp, zeros)
    u3 = _tc_mid(s2, u2, hist, b2r, g2r, be2r, W3)
    s3 = _sc_edges(u3, srcp, dstp, zeros)
    outp = _tc_fin(s3, u3, hist, b3r)
    return outp[:N]


# speedup vs baseline: 3.1736x; 1.0048x over previous
import functools

import jax
import jax.numpy as jnp
from jax import lax
from jax.experimental import pallas as pl
from jax.experimental.pallas import tpu as pltpu
from jax.experimental.pallas import tpu_sc as plsc

N = 10000
E = 320000
D = 128
BN_SCALE = float(1.0 / (1.0 + 1e-5) ** 0.5)

NC, NS = 2, 16
TILES = NC * NS
CH = 128
NCH = 80
SB = 16
EP = TILES * NCH * CH
NP = 10240
RPT = NP // NS

_mesh = plsc.VectorSubcoreMesh(core_axis_name="c", subcore_axis_name="s")


def _deg_body(dst_hbm, zeros_hbm, ones_hbm, out_hbm, idx_d, ones_v, acc,
              sem0, sem1):
    c = lax.axis_index("c")
    s = lax.axis_index("s")
    w = c * NS + s
    pltpu.sync_copy(zeros_hbm.at[pl.ds(s * RPT, RPT)], acc.at[pl.ds(s * RPT, RPT)])
    pltpu.sync_copy(ones_hbm, ones_v)
    plsc.subcore_barrier()
    sems = (sem0, sem1)

    def sblock(b, carry):
        pltpu.sync_copy(dst_hbm.at[w].at[b], idx_d)
        cs = [None] * SB
        for j in range(SB):
            if j >= 2:
                cs[j - 2].wait()
            cs[j] = pltpu.async_copy(ones_v, acc.at[idx_d.at[j]],
                                     sems[j % 2], add=True)
        cs[SB - 2].wait()
        cs[SB - 1].wait()
        return carry

    lax.fori_loop(0, NCH // SB, sblock, 0)
    plsc.subcore_barrier()
    pltpu.sync_copy(acc.at[pl.ds(s * RPT, RPT)], out_hbm.at[c, pl.ds(s * RPT, RPT)])


_sc_deg = pl.kernel(
    _deg_body,
    out_type=jax.ShapeDtypeStruct((NC, NP, D), jnp.float32),
    mesh=_mesh,
    scratch_types=[
        pltpu.VMEM((SB, CH), jnp.int32),
        pltpu.VMEM((CH, D), jnp.float32),
        pltpu.VMEM_SHARED((NP, D), jnp.float32),
        pltpu.SemaphoreType.DMA,
        pltpu.SemaphoreType.DMA,
    ],
)


def _edge_body(u_hbm, src_hbm, dst_hbm, zeros_hbm, out_hbm,
               idx_s, idx_d, rows0, rows1, acc, gsem0, gsem1, ssem0, ssem1):
    c = lax.axis_index("c")
    s = lax.axis_index("s")
    w = c * NS + s
    pltpu.sync_copy(zeros_hbm.at[pl.ds(s * RPT, RPT)], acc.at[pl.ds(s * RPT, RPT)])
    plsc.subcore_barrier()

    rows = (rows0, rows1)
    gsems = (gsem0, gsem1)
    ssems = (ssem0, ssem1)

    def sblock(b, carry):
        pltpu.sync_copy(src_hbm.at[w].at[b], idx_s)
        pltpu.sync_copy(dst_hbm.at[w].at[b], idx_d)
        cg = [None] * SB
        cs = [None] * SB
        cg[0] = pltpu.async_copy(u_hbm.at[idx_s.at[0]], rows[0], gsems[0])
        for j in range(SB):
            if j >= 1:
                cs[j - 1].wait()
            if j + 1 < SB:
                cg[j + 1] = pltpu.async_copy(
                    u_hbm.at[idx_s.at[j + 1]], rows[(j + 1) % 2], gsems[(j + 1) % 2])
            cg[j].wait()
            cs[j] = pltpu.async_copy(
                rows[j % 2], acc.at[idx_d.at[j]], ssems[j % 2], add=True)
        cs[SB - 1].wait()
        return carry

    lax.fori_loop(0, NCH // SB, sblock, 0)
    plsc.subcore_barrier()
    pltpu.sync_copy(acc.at[pl.ds(s * RPT, RPT)], out_hbm.at[c, pl.ds(s * RPT, RPT)])


_sc_edges = pl.kernel(
    _edge_body,
    out_type=jax.ShapeDtypeStruct((NC, NP, D), jnp.float32),
    mesh=_mesh,
    scratch_types=[
        pltpu.VMEM((SB, CH), jnp.int32),
        pltpu.VMEM((SB, CH), jnp.int32),
        pltpu.VMEM((CH, D), jnp.float32),
        pltpu.VMEM((CH, D), jnp.float32),
        pltpu.VMEM_SHARED((NP, D), jnp.float32),
        pltpu.SemaphoreType.DMA,
        pltpu.SemaphoreType.DMA,
        pltpu.SemaphoreType.DMA,
        pltpu.SemaphoreType.DMA,
    ],
)


BR = 1024


def _pre_body(x_ref, w_ref, h_ref, o_ref, d_ref):
    deg = 1.0 + h_ref[0, :, 0] + h_ref[1, :, 0]
    dinv = lax.rsqrt(deg)[:, None]
    xw = jnp.dot(x_ref[...], w_ref[...], preferred_element_type=jnp.float32)
    o_ref[...] = xw * dinv
    d_ref[...] = dinv


def _mid_body(s_ref, u_ref, d_ref, b_ref, g_ref, be_ref, w_ref, o_ref):
    dinv = d_ref[...]
    pre = dinv * (s_ref[0] + s_ref[1] + u_ref[...]) + b_ref[...]
    h = jnp.maximum(pre * (g_ref[...] * BN_SCALE) + be_ref[...], 0.0)
    o_ref[...] = jnp.dot(h, w_ref[...], preferred_element_type=jnp.float32) * dinv


def _fin_body(s_ref, u_ref, d_ref, b_ref, o_ref):
    o_ref[...] = d_ref[...] * (s_ref[0] + s_ref[1] + u_ref[...]) + b_ref[...]


_GRID = NP // BR
_bs_rows = pl.BlockSpec((BR, D), lambda i: (i, 0))
_bs_part = pl.BlockSpec((NC, BR, D), lambda i: (0, i, 0))
_bs_hist = pl.BlockSpec((NC, BR, D), lambda i: (0, i, 0))
_bs_dinv = pl.BlockSpec((BR, 1), lambda i: (i, 0))
_bs_w = pl.BlockSpec((D, D), lambda i: (0, 0))
_bs_vec = pl.BlockSpec((1, D), lambda i: (0, 0))

_tc_pre = pl.pallas_call(
    _pre_body,
    grid=(_GRID,),
    in_specs=[_bs_rows, _bs_w, _bs_hist],
    out_specs=[_bs_rows, _bs_dinv],
    out_shape=[jax.ShapeDtypeStruct((NP, D), jnp.float32),
               jax.ShapeDtypeStruct((NP, 1), jnp.float32)],
)

_tc_mid = pl.pallas_call(
    _mid_body,
    grid=(_GRID,),
    in_specs=[_bs_part, _bs_rows, _bs_dinv, _bs_vec, _bs_vec, _bs_vec, _bs_w],
    out_specs=_bs_rows,
    out_shape=jax.ShapeDtypeStruct((NP, D), jnp.float32),
)

_tc_fin = pl.pallas_call(
    _fin_body,
    grid=(_GRID,),
    in_specs=[_bs_part, _bs_rows, _bs_dinv, _bs_vec],
    out_specs=_bs_rows,
    out_shape=jax.ShapeDtypeStruct((NP, D), jnp.float32),
)


@jax.jit
def kernel(x, edge_index, W1, b1, g1, be1, W2, b2, g2, be2, W3, b3):
    f32 = jnp.float32
    xp = jnp.zeros((NP, D), f32).at[:N].set(x)
    pad = N + jnp.arange(EP - E, dtype=jnp.int32) % (NP - N)
    srcp = jnp.concatenate([edge_index[0], pad]).reshape(TILES, NCH // SB, SB, CH)
    dstp = jnp.concatenate([edge_index[1], pad]).reshape(TILES, NCH // SB, SB, CH)
    zeros = jnp.zeros((NP, D), f32)
    onesr = jnp.ones((CH, D), f32)
    b1r, g1r, be1r = b1.reshape(1, D), g1.reshape(1, D), be1.reshape(1, D)
    b2r, g2r, be2r = b2.reshape(1, D), g2.reshape(1, D), be2.reshape(1, D)
    b3r = b3.reshape(1, D)

    hist = _sc_deg(dstp, zeros, onesr)
    u1, dinv = _tc_pre(xp, W1, hist)
    s1 = _sc_edges(u1, srcp, dstp, zeros)
    u2 = _tc_mid(s1, u1, dinv, b1r, g1r, be1r, W2)
    s2 = _sc_edges(u2, srcp, dstp, zeros)
    u3 = _tc_mid(s2, u2, dinv, b2r, g2r, be2r, W3)
    s3 = _sc_edges(u3, srcp, dstp, zeros)
    outp = _tc_fin(s3, u3, dinv, b3r)
    return outp[:N]
